# baseline (speedup 1.0000x reference)
import functools

import jax
import jax.numpy as jnp
from jax import lax
from jax.experimental import pallas as pl
from jax.experimental.pallas import tpu as pltpu
from jax.experimental.pallas import tpu_sc as plsc

_NUM_CORES = 2
_NUM_SUBCORES = 16
_NUM_WORKERS = _NUM_CORES * _NUM_SUBCORES
_LANES = 16
_PACK = 4
_SUB = 128


@functools.lru_cache(maxsize=None)
def _make_gather(K, B, D):
    chunk = B // _NUM_WORKERS
    assert chunk * _NUM_WORKERS == B
    nsub = chunk // _SUB
    assert nsub * _SUB == chunk and K % 2 == 0

    mesh = plsc.VectorSubcoreMesh(core_axis_name="c", subcore_axis_name="s")

    @functools.partial(
        pl.kernel,
        mesh=mesh,
        out_type=jax.ShapeDtypeStruct((K, D, B), jnp.float32),
        scratch_types=[
            pltpu.VMEM((K, nsub, _SUB), jnp.int32),
            pltpu.VMEM((K, nsub, _SUB), jnp.int32),
            pltpu.VMEM((2, _SUB, _PACK * D), jnp.float32),
            pltpu.VMEM((2, D, chunk), jnp.float32),
            pltpu.SemaphoreType.DMA,
            pltpu.SemaphoreType.DMA,
            pltpu.SemaphoreType.DMA,
            pltpu.SemaphoreType.DMA,
        ],
        compiler_params=pltpu.CompilerParams(
            use_tc_tiling_on_sc=True, needs_layout_passes=False),
    )
    def body(idx_hbm, t4_hbm, out_hbm, idx_v, div_v, rows_v, t_v,
             g0, g1, w0, w1):
        gsems = (g0, g1)
        wsems = (w0, w1)
        wid = lax.axis_index("s") * _NUM_CORES + lax.axis_index("c")
        b0 = wid * chunk

        for q in range(nsub):
            pltpu.sync_copy(
                idx_hbm.at[:, pl.ds(b0 + q * _SUB, _SUB)], idx_v.at[:, q])
        for kk in range(K):
            for q in range(nsub):
                @plsc.parallel_loop(0, _SUB // _LANES, unroll=4)
                def _div(j):
                    v = idx_v[kk, q, pl.ds(j * _LANES, _LANES)]
                    div_v[kk, q, pl.ds(j * _LANES, _LANES)] = (
                        lax.shift_right_logical(v, 2))

        def gather_copy(k, q, buf):
            return pltpu.make_async_copy(
                t4_hbm.at[div_v.at[k, q]], rows_v.at[buf], gsems[buf])

        def write_copy(k, p):
            return pltpu.make_async_copy(
                t_v.at[p], out_hbm.at[k, :, pl.ds(b0, chunk)], wsems[p])

        def sub_step(k, q, p):
            buf = q % 2
            if q + 1 < nsub:
                gather_copy(k, q + 1, buf ^ 1).start()
            else:
                @pl.when(k + 1 < K)
                def _pref():
                    gather_copy(k + 1, 0, buf ^ 1).start()

            gather_copy(k, q, buf).wait()
            if q == 0:
                @pl.when(k >= 2)
                def _drain():
                    write_copy(k - 2, p).wait()

            rows_p = rows_v.at[buf]
            t_p = t_v.at[p]
            lane = lax.iota(jnp.int32, _LANES)

            @plsc.parallel_loop(0, _SUB // _LANES, unroll=4)
            def transpose_j(j):
                rows16 = j * _LANES + lane
                idx16 = idx_v[k, q, pl.ds(j * _LANES, _LANES)]
                base = (idx16 & 3) * D
                for d in range(D):
                    t_p[d, pl.ds(q * _SUB + j * _LANES, _LANES)] = (
                        plsc.load_gather(rows_p, [rows16, base + d]))

            if q == nsub - 1:
                write_copy(k, p).start()

        gather_copy(0, 0, 0).start()

        def outer(i, carry):
            for par in range(2):
                k = i * 2 + par
                for q in range(nsub):
                    sub_step(k, q, par)
            return carry

        lax.fori_loop(0, K // 2, outer, 0)
        write_copy(K - 2, 0).wait()
        write_copy(K - 1, 1).wait()

    return body


def kernel(indices, table):
    Bq, K = indices.shape
    V, D = table.shape
    idxT = indices.T.astype(jnp.int32)
    t4 = table.reshape(V // _PACK, _PACK * D)
    res = _make_gather(K, Bq, D)(idxT, t4)
    return res.transpose(2, 0, 1)


# speedup vs baseline: 1.0344x; 1.0344x over previous
import functools

import jax
import jax.numpy as jnp
from jax import lax
from jax.experimental import pallas as pl
from jax.experimental.pallas import tpu as pltpu
from jax.experimental.pallas import tpu_sc as plsc

_NUM_CORES = 2
_NUM_SUBCORES = 16
_NUM_WORKERS = _NUM_CORES * _NUM_SUBCORES
_LANES = 16
_PACK = 4
_SUB = 128


@functools.lru_cache(maxsize=None)
def _make_gather(K, B, D):
    chunk = B // _NUM_WORKERS
    assert chunk * _NUM_WORKERS == B
    nsub = chunk // _SUB
    assert nsub * _SUB == chunk and K % 2 == 0

    mesh = plsc.VectorSubcoreMesh(core_axis_name="c", subcore_axis_name="s")

    @functools.partial(
        pl.kernel,
        mesh=mesh,
        out_type=jax.ShapeDtypeStruct((K, D, B), jnp.float32),
        scratch_types=[
            pltpu.VMEM((K, nsub, _SUB), jnp.int32),
            pltpu.VMEM((K, nsub, _SUB), jnp.int32),
            pltpu.VMEM((4, _SUB, _PACK * D), jnp.float32),
            pltpu.VMEM((2, D, chunk), jnp.float32),
            pltpu.SemaphoreType.DMA,
            pltpu.SemaphoreType.DMA,
            pltpu.SemaphoreType.DMA,
            pltpu.SemaphoreType.DMA,
            pltpu.SemaphoreType.DMA,
            pltpu.SemaphoreType.DMA,
        ],
        compiler_params=pltpu.CompilerParams(
            use_tc_tiling_on_sc=True, needs_layout_passes=False),
    )
    def body(idx_hbm, t4_hbm, out_hbm, idx_v, div_v, rows_v, t_v,
             g0, g1, g2, g3, w0, w1):
        gsems = (g0, g1, g2, g3)
        wsems = (w0, w1)
        wid = lax.axis_index("s") * _NUM_CORES + lax.axis_index("c")
        b0 = wid * chunk

        for q in range(nsub):
            pltpu.sync_copy(
                idx_hbm.at[:, pl.ds(b0 + q * _SUB, _SUB)], idx_v.at[:, q])
        for kk in range(K):
            for q in range(nsub):
                @plsc.parallel_loop(0, _SUB // _LANES, unroll=4)
                def _div(j):
                    v = idx_v[kk, q, pl.ds(j * _LANES, _LANES)]
                    div_v[kk, q, pl.ds(j * _LANES, _LANES)] = (
                        lax.shift_right_logical(v, 2))

        def gather_copy(k, q, buf):
            return pltpu.make_async_copy(
                t4_hbm.at[div_v.at[k, q]], rows_v.at[buf], gsems[buf])

        def write_copy(k, p):
            return pltpu.make_async_copy(
                t_v.at[p], out_hbm.at[k, :, pl.ds(b0, chunk)], wsems[p])

        def sub_step(k, q, p):
            buf = q
            qp = q + 3
            kn, qn = k + qp // nsub, qp % nsub
            if qp // nsub == 0:
                gather_copy(k, qn, qn).start()
            else:
                @pl.when(kn < K)
                def _pref():
                    gather_copy(kn, qn, qn).start()

            gather_copy(k, q, buf).wait()
            if q == 0:
                @pl.when(k >= 2)
                def _drain():
                    write_copy(k - 2, p).wait()

            rows_p = rows_v.at[buf]
            t_p = t_v.at[p]
            lane = lax.iota(jnp.int32, _LANES)

            @plsc.parallel_loop(0, _SUB // _LANES, unroll=2)
            def transpose_j(j):
                rows16 = j * _LANES + lane
                idx16 = idx_v[k, q, pl.ds(j * _LANES, _LANES)]
                base = (idx16 & 3) * D
                for d in range(D):
                    t_p[d, pl.ds(q * _SUB + j * _LANES, _LANES)] = (
                        plsc.load_gather(rows_p, [rows16, base + d]))

            if q == nsub - 1:
                write_copy(k, p).start()

        gather_copy(0, 0, 0).start()
        gather_copy(0, 1, 1).start()
        gather_copy(0, 2, 2).start()

        def outer(i, carry):
            for par in range(2):
                k = i * 2 + par
                for q in range(nsub):
                    sub_step(k, q, par)
            return carry

        lax.fori_loop(0, K // 2, outer, 0)
        write_copy(K - 2, 0).wait()
        write_copy(K - 1, 1).wait()

    return body


def kernel(indices, table):
    Bq, K = indices.shape
    V, D = table.shape
    idxT = indices.T.astype(jnp.int32)
    t4 = table.reshape(V // _PACK, _PACK * D)
    res = _make_gather(K, Bq, D)(idxT, t4)
    return res.transpose(2, 0, 1)


# speedup vs baseline: 1.0355x; 1.0011x over previous
import functools

import jax
import jax.numpy as jnp
from jax import lax
from jax.experimental import pallas as pl
from jax.experimental.pallas import tpu as pltpu
from jax.experimental.pallas import tpu_sc as plsc

_NUM_CORES = 2
_NUM_SUBCORES = 16
_NUM_WORKERS = _NUM_CORES * _NUM_SUBCORES
_LANES = 16
_PACK = 4
_SUB = 128


@functools.lru_cache(maxsize=None)
def _make_gather(K, B, D):
    chunk = B // _NUM_WORKERS
    assert chunk * _NUM_WORKERS == B
    nsub = chunk // _SUB
    assert nsub * _SUB == chunk and K % 2 == 0

    mesh = plsc.VectorSubcoreMesh(core_axis_name="c", subcore_axis_name="s")

    @functools.partial(
        pl.kernel,
        mesh=mesh,
        out_type=jax.ShapeDtypeStruct((K, D, B), jnp.float32),
        scratch_types=[
            pltpu.VMEM((K, nsub, _SUB), jnp.int32),
            pltpu.VMEM((K, nsub, _SUB), jnp.int32),
            pltpu.VMEM((4, _SUB, _PACK * D), jnp.float32),
            pltpu.VMEM((2, D, chunk), jnp.float32),
            pltpu.SemaphoreType.DMA,
            pltpu.SemaphoreType.DMA,
            pltpu.SemaphoreType.DMA,
            pltpu.SemaphoreType.DMA,
            pltpu.SemaphoreType.DMA,
            pltpu.SemaphoreType.DMA,
        ],
        compiler_params=pltpu.CompilerParams(
            use_tc_tiling_on_sc=True, needs_layout_passes=False),
    )
    def body(idx_hbm, t4_hbm, out_hbm, idx_v, div_v, rows_v, t_v,
             g0, g1, g2, g3, w0, w1):
        gsems = (g0, g1, g2, g3)
        wsems = (w0, w1)
        wid = lax.axis_index("s") * _NUM_CORES + lax.axis_index("c")
        b0 = wid * chunk

        for q in range(nsub):
            pltpu.sync_copy(
                idx_hbm.at[:, pl.ds(b0 + q * _SUB, _SUB)], idx_v.at[:, q])
        for kk in range(K):
            for q in range(nsub):
                @plsc.parallel_loop(0, _SUB // _LANES, unroll=4)
                def _div(j):
                    v = idx_v[kk, q, pl.ds(j * _LANES, _LANES)]
                    div_v[kk, q, pl.ds(j * _LANES, _LANES)] = (
                        lax.shift_right_logical(v, 2))

        def gather_copy(k, q, buf):
            return pltpu.make_async_copy(
                t4_hbm.at[div_v.at[k, q]], rows_v.at[buf], gsems[buf])

        def write_copy(k, p):
            return pltpu.make_async_copy(
                t_v.at[p], out_hbm.at[k, :, pl.ds(b0, chunk)], wsems[p])

        def sub_step(k, q, p):
            buf = q
            qp = q + 3
            kn, qn = k + qp // nsub, qp % nsub
            if qp // nsub == 0:
                gather_copy(k, qn, qn).start()
            else:
                @pl.when(kn < K)
                def _pref():
                    gather_copy(kn, qn, qn).start()

            gather_copy(k, q, buf).wait()
            if q == 0:
                @pl.when(k >= 2)
                def _drain():
                    write_copy(k - 2, p).wait()

            rows_p = rows_v.at[buf]
            t_p = t_v.at[p]
            lane = lax.iota(jnp.int32, _LANES)

            @plsc.parallel_loop(0, _SUB // _LANES, unroll=1)
            def transpose_j(j):
                rows16 = j * _LANES + lane
                idx16 = idx_v[k, q, pl.ds(j * _LANES, _LANES)]
                base = (idx16 & 3) * D
                for d in range(D):
                    t_p[d, pl.ds(q * _SUB + j * _LANES, _LANES)] = (
                        plsc.load_gather(rows_p, [rows16, base + d]))

            if q == nsub - 1:
                write_copy(k, p).start()

        gather_copy(0, 0, 0).start()
        gather_copy(0, 1, 1).start()
        gather_copy(0, 2, 2).start()

        def outer(i, carry):
            for par in range(2):
                k = i * 2 + par
                for q in range(nsub):
                    sub_step(k, q, par)
            return carry

        lax.fori_loop(0, K // 2, outer, 0)
        write_copy(K - 2, 0).wait()
        write_copy(K - 1, 1).wait()

    return body


def kernel(indices, table):
    Bq, K = indices.shape
    V, D = table.shape
    idxT = indices.T.astype(jnp.int32)
    t4 = table.reshape(V // _PACK, _PACK * D)
    res = _make_gather(K, Bq, D)(idxT, t4)
    return res.transpose(2, 0, 1)
